# baseline (speedup 1.0000x reference)
import jax
import jax.numpy as jnp
from jax import lax
from jax.experimental import pallas as pl
from jax.experimental.pallas import tpu as pltpu
from jax.experimental.pallas import tpu_sc as plsc

N = 5000
K = 80
IMG = 640.0
NMS_THRESH = 0.5
NPAD = 5120
BLK = 128
NBLK = NPAD // BLK
NW = 32
PW = 12544
TOT = NW * PW
ROWS = TOT // 128
RB = 392
HIGH = jax.lax.Precision.HIGHEST


def _a_body(x_ref, sig_ref, posm_ref, a_ref):
    x = x_ref[...]
    sig_ref[0:3125, :] = jax.nn.sigmoid(x)
    sig_ref[3125:ROWS, :] = jnp.zeros((ROWS - 3125, 128), jnp.float32)
    sb = lax.bitcast_convert_type(sig_ref[...], jnp.int32)

    def cond(c):
        return c[1] - c[0] > 1

    def body(c):
        lo, hi = c
        mid = (lo + hi) // 2
        cnt = jnp.sum(jnp.where(sb > mid, 1, 0))
        return jnp.where(cnt >= N, mid, lo), jnp.where(cnt >= N, hi, mid)

    _, pstar = lax.while_loop(cond, body, (jnp.int32(0), jnp.int32(0x3F800000)))

    gm = jnp.where(sb > pstar, 1.0, 0.0)
    em = jnp.where(sb == pstar, 1.0, 0.0)
    ri = lax.broadcasted_iota(jnp.int32, (BLK, BLK), 0)
    ci = lax.broadcasted_iota(jnp.int32, (BLK, BLK), 1)
    mstrict = jnp.where(ri < ci, 1.0, 0.0)
    gex = lax.dot_general(gm, mstrict, (((1,), (0,)), ((), ())),
                          preferred_element_type=jnp.float32)
    eex = lax.dot_general(em, mstrict, (((1,), (0,)), ((), ())),
                          preferred_element_type=jnp.float32)
    a_ref[:, 0:1] = jnp.sum(gm, axis=1, keepdims=True)
    a_ref[:, 1:2] = jnp.sum(em, axis=1, keepdims=True)

    rb_r = lax.broadcasted_iota(jnp.int32, (RB, RB), 0)
    rb_c = lax.broadcasted_iota(jnp.int32, (RB, RB), 1)
    lstrict = jnp.where(rb_c < rb_r, 1.0, 0.0)
    rowi = lax.broadcasted_iota(jnp.int32, (RB, 1), 0)

    def rp_step(b, c):
        cg, ce = c
        gt = a_ref[pl.ds(b * RB, RB), 0:1]
        et = a_ref[pl.ds(b * RB, RB), 1:2]
        a_ref[pl.ds(b * RB, RB), 2:3] = lax.dot_general(
            lstrict, gt, (((1,), (0,)), ((), ())),
            preferred_element_type=jnp.float32) + cg
        a_ref[pl.ds(b * RB, RB), 3:4] = lax.dot_general(
            lstrict, et, (((1,), (0,)), ((), ())),
            preferred_element_type=jnp.float32) + ce
        return cg + jnp.sum(gt), ce + jnp.sum(et)

    m, _ = lax.fori_loop(0, ROWS // RB, rp_step, (jnp.float32(0.0), jnp.float32(0.0)))
    r = jnp.float32(N) - m

    pos_g = a_ref[:, 2:3] + gex
    pos_e = a_ref[:, 3:4] + eex
    pos = jnp.where(gm > 0.5, pos_g,
                    jnp.where(em > 0.5,
                              jnp.where(pos_e < r, m + pos_e, -1.0),
                              -1.0))
    posm_ref[...] = pos.astype(jnp.int32)


def _run_a(x2d, interpret=False):
    return pl.pallas_call(
        _a_body,
        out_shape=(jax.ShapeDtypeStruct((ROWS, 128), jnp.float32),
                   jax.ShapeDtypeStruct((ROWS, 128), jnp.int32)),
        scratch_shapes=[pltpu.VMEM((ROWS, 8), jnp.float32)],
        interpret=interpret,
    )(x2d)


def _b_body(scores_hbm, posm_hbm, vals_hbm, idx_hbm, sc_v, pos_v, lv, li):
    w = lax.axis_index("s") * 2 + lax.axis_index("c")
    base = w * PW
    pltpu.sync_copy(scores_hbm.at[pl.ds(base, PW)], sc_v)
    pltpu.sync_copy(posm_hbm.at[pl.ds(base, PW)], pos_v)
    lanes = lax.iota(jnp.int32, 16)

    def zstep(i, c):
        lv[pl.ds(i * 16, 16)] = jnp.zeros((16,), jnp.float32)
        li[pl.ds(i * 16, 16)] = jnp.zeros((16,), jnp.int32)
        return c

    lax.fori_loop(0, NPAD // 16, zstep, jnp.int32(0))

    def step(i, c):
        v = sc_v[pl.ds(i * 16, 16)]
        p = pos_v[pl.ds(i * 16, 16)]
        flat = base + i * 16 + lanes
        msk = p >= 0
        pc = jnp.where(msk, p, 0)
        plsc.store_scatter(lv, [pc], v, mask=msk)
        plsc.store_scatter(li, [pc], flat, mask=msk)
        return c

    lax.fori_loop(0, PW // 16, step, jnp.int32(0))
    pltpu.sync_copy(lv, vals_hbm.at[w])
    pltpu.sync_copy(li, idx_hbm.at[w])


def _run_b(scores_flat, posm_flat):
    mesh = plsc.VectorSubcoreMesh(core_axis_name="c", subcore_axis_name="s")
    f = pl.kernel(
        _b_body,
        out_type=(jax.ShapeDtypeStruct((NW, NPAD), jnp.float32),
                  jax.ShapeDtypeStruct((NW, NPAD), jnp.int32)),
        mesh=mesh,
        compiler_params=pltpu.CompilerParams(needs_layout_passes=False),
        scratch_types=[
            pltpu.VMEM((PW,), jnp.float32),
            pltpu.VMEM((PW,), jnp.int32),
            pltpu.VMEM((NPAD,), jnp.float32),
            pltpu.VMEM((NPAD,), jnp.int32),
        ],
    )
    return f(scores_flat, posm_flat)


def _row2col(v, eye):
    return jnp.sum(eye * v, axis=1, keepdims=True)


def _col2row(v, eye):
    return jnp.sum(eye * v, axis=0, keepdims=True)


def _c_body(vals2_ref, idx2_ref, boxpay_ref, dets_ref, lab_ref, t_ref, s_ref):
    col = lax.broadcasted_iota(jnp.int32, (1, NPAD), 1)
    ri = lax.broadcasted_iota(jnp.int32, (BLK, BLK), 0)
    ci = lax.broadcasted_iota(jnp.int32, (BLK, BLK), 1)
    eye = jnp.where(ri == ci, 1.0, 0.0)
    real = col < N

    vrow = jnp.sum(vals2_ref[...], axis=0, keepdims=True)
    irow = jnp.sum(idx2_ref[...], axis=0, keepdims=True)
    vrow = jnp.where(real, vrow, 0.0)
    irow = jnp.where(real, irow, 0)
    l2 = (irow >> 16).astype(jnp.float32)
    l1 = ((irow >> 8) & 255).astype(jnp.float32)
    l0 = (irow & 255).astype(jnp.float32)
    s_ref[5:6, :] = vrow
    colf = col.astype(jnp.float32)

    def rank_step(i, c):
        base = i * BLK
        vb = _row2col(s_ref[5:6, pl.ds(base, BLK)], eye)
        jg = (base + lax.broadcasted_iota(jnp.int32, (BLK, 1), 0)).astype(jnp.float32)
        gt = jnp.where(vrow > vb, 1.0, 0.0)
        eq = jnp.where(vrow == vb, 1.0, 0.0) * jnp.where(colf < jg, 1.0, 0.0)
        rk = jnp.sum(gt + eq, axis=1, keepdims=True)
        t_ref[6:7, pl.ds(base, BLK)] = _col2row(rk, eye)
        return c

    lax.fori_loop(0, NBLK, rank_step, jnp.int32(0))

    rank_row = t_ref[6:7, :]
    boxpay = boxpay_ref[...]
    payt = jnp.concatenate([vrow, l2, l1, l0], axis=0)

    def perm_step(i, c):
        base = i * BLK
        jout = (base + lax.broadcasted_iota(jnp.int32, (BLK, 1), 0)).astype(jnp.float32)
        oh = jnp.where(rank_row == jout, 1.0, 0.0)
        rec = lax.dot_general(oh, payt, (((1,), (1,)), ((), ())),
                              precision=HIGH,
                              preferred_element_type=jnp.float32)
        sv = rec[:, 0:1]
        idx = (rec[:, 1:2] * 65536.0 + rec[:, 2:3] * 256.0
               + rec[:, 3:4]).astype(jnp.int32)
        prop = idx // K
        label = idx - prop * K
        oh2 = jnp.where(colf == prop.astype(jnp.float32), 1.0, 0.0)
        bl = lax.dot_general(oh2, boxpay, (((1,), (0,)), ((), ())),
                             precision=HIGH,
                             preferred_element_type=jnp.float32)

        def coord(cix):
            l3i = bl[:, 4 * cix + 0:4 * cix + 1].astype(jnp.int32)
            l2i = bl[:, 4 * cix + 1:4 * cix + 2].astype(jnp.int32)
            l1i = bl[:, 4 * cix + 2:4 * cix + 3].astype(jnp.int32)
            l0i = bl[:, 4 * cix + 3:4 * cix + 4].astype(jnp.int32)
            bits = l3i * 16777216 + l2i * 65536 + l1i * 256 + l0i
            return lax.bitcast_convert_type(bits, jnp.float32)

        cx, cy, bw, bh = coord(0), coord(1), coord(2), coord(3)
        x1 = (cx - 0.5 * bw) * IMG
        y1 = (cy - 0.5 * bh) * IMG
        x2 = (cx + 0.5 * bw) * IMG
        y2 = (cy + 0.5 * bh) * IMG
        t_ref[0:1, pl.ds(base, BLK)] = _col2row(x1, eye)
        t_ref[1:2, pl.ds(base, BLK)] = _col2row(y1, eye)
        t_ref[2:3, pl.ds(base, BLK)] = _col2row(x2, eye)
        t_ref[3:4, pl.ds(base, BLK)] = _col2row(y2, eye)
        t_ref[4:5, pl.ds(base, BLK)] = _col2row(label.astype(jnp.float32), eye)
        t_ref[5:6, pl.ds(base, BLK)] = _col2row(sv, eye)
        return c

    lax.fori_loop(0, NBLK, perm_step, jnp.int32(0))

    x1 = t_ref[0:1, :]
    y1 = t_ref[1:2, :]
    x2 = t_ref[2:3, :]
    y2 = t_ref[3:4, :]
    labels_f = t_ref[4:5, :]
    vals_s = t_ref[5:6, :]
    neg = jnp.float32(-1e30)
    mx = jnp.max(jnp.where(real, jnp.maximum(jnp.maximum(x1, y1),
                                             jnp.maximum(x2, y2)), neg))
    maxc = mx + 1.0
    off = labels_f * maxc
    ox1 = x1 + off
    oy1 = y1 + off
    ox2 = x2 + off
    oy2 = y2 + off
    area = (ox2 - ox1) * (oy2 - oy1)
    s_ref[0:1, :] = ox1
    s_ref[1:2, :] = oy1
    s_ref[2:3, :] = ox2
    s_ref[3:4, :] = oy2
    s_ref[4:5, :] = area
    t_ref[7:8, :] = jnp.zeros((1, NPAD), jnp.float32)

    def block_step(i, carry):
        base = i * BLK
        rx1 = s_ref[0:1, pl.ds(base, BLK)]
        ry1 = s_ref[1:2, pl.ds(base, BLK)]
        rx2 = s_ref[2:3, pl.ds(base, BLK)]
        ry2 = s_ref[3:4, pl.ds(base, BLK)]
        rar = s_ref[4:5, pl.ds(base, BLK)]
        bx1 = _row2col(rx1, eye)
        by1 = _row2col(ry1, eye)
        bx2 = _row2col(rx2, eye)
        by2 = _row2col(ry2, eye)
        bar = _row2col(rar, eye)

        def over_f(ax1, ay1, ax2, ay2, aar):
            ix1 = jnp.maximum(bx1, ax1)
            iy1 = jnp.maximum(by1, ay1)
            ix2 = jnp.minimum(bx2, ax2)
            iy2 = jnp.minimum(by2, ay2)
            inter = jnp.maximum(ix2 - ix1, 0.0) * jnp.maximum(iy2 - iy1, 0.0)
            iou = inter / (bar + aar - inter + 1e-9)
            return jnp.where(iou > NMS_THRESH, 1.0, 0.0)

        over_all = over_f(ox1, oy1, ox2, oy2, area)
        keep = t_ref[7:8, :]
        prior = jnp.where(col < base, keep, 0.0)
        supp0 = jnp.sum(over_all * prior, axis=1, keepdims=True)
        e_col = 1.0 - jnp.minimum(supp0, 1.0)
        tri = jnp.where(ri < ci, 1.0, 0.0)
        ob = over_f(rx1, ry1, rx2, ry2, rar) * tri

        def fp_cond(c):
            return c[1] > 0.5

        def fp_body(c):
            kb, _ = c
            sup = jnp.sum(ob * kb, axis=0, keepdims=True)
            sup_col = _row2col(jnp.minimum(sup, 1.0), eye)
            nk = e_col * (1.0 - jnp.minimum(sup_col, 1.0))
            return nk, jnp.sum(jnp.abs(nk - kb))

        kb, _ = lax.while_loop(fp_cond, fp_body, (e_col, jnp.float32(1.0)))
        t_ref[7:8, pl.ds(base, BLK)] = _col2row(kb, eye)
        return carry

    lax.fori_loop(0, NBLK, block_step, jnp.int32(0))
    keep = jnp.where(real, t_ref[7:8, :], 0.0)

    c0 = jnp.zeros((1, NPAD), jnp.float32)
    dets_ref[0:1, :] = jnp.clip(x1, 0.0, IMG) * keep
    dets_ref[1:2, :] = jnp.clip(y1, 0.0, IMG) * keep
    dets_ref[2:3, :] = jnp.clip(x2, 0.0, IMG) * keep
    dets_ref[3:4, :] = jnp.clip(y2, 0.0, IMG) * keep
    dets_ref[4:5, :] = vals_s * keep
    dets_ref[5:6, :] = c0
    dets_ref[6:7, :] = c0
    dets_ref[7:8, :] = c0
    lab_ref[0:1, :] = labels_f


def _run_c(vals2, idx2, boxpay, interpret=False):
    return pl.pallas_call(
        _c_body,
        out_shape=(jax.ShapeDtypeStruct((8, NPAD), jnp.float32),
                   jax.ShapeDtypeStruct((1, NPAD), jnp.float32)),
        scratch_shapes=[pltpu.VMEM((8, NPAD), jnp.float32),
                        pltpu.VMEM((8, NPAD), jnp.float32)],
        interpret=interpret,
    )(vals2, idx2, boxpay)


def _boxpay(box_pred):
    bbits = lax.bitcast_convert_type(box_pred[0], jnp.int32)
    limbs = [((bbits[:, c] >> s) & 255).astype(jnp.float32)
             for c in range(4) for s in (24, 16, 8, 0)]
    return jnp.pad(jnp.stack(limbs, axis=1), ((0, NPAD - N), (0, 0)))


@jax.jit
def _pipeline(box_cls, box_pred):
    x2d = box_cls[0].reshape(3125, 128)
    sig, posm = _run_a(x2d)
    vals2, idx2 = _run_b(sig.reshape(TOT), posm.reshape(TOT))
    dets8, lab = _run_c(vals2, idx2, _boxpay(box_pred))
    dets = dets8[:5, :N].T
    labels = lab[0, :N].astype(jnp.int32)
    return dets, labels


def kernel(box_cls, box_pred):
    return _pipeline(box_cls, box_pred)


# speedup vs baseline: 1.6709x; 1.6709x over previous
import jax
import jax.numpy as jnp
from jax import lax
from jax.experimental import pallas as pl
from jax.experimental.pallas import tpu as pltpu
from jax.experimental.pallas import tpu_sc as plsc

N = 5000
K = 80
IMG = 640.0
NMS_THRESH = 0.5
NPAD = 5120
BLK = 128
CBLK = 256
CNB = NPAD // CBLK
NBLK = NPAD // BLK
NW = 32
PW = 12544
TOT = NW * PW
ROWS = TOT // 128
RB = 392
HIGH = jax.lax.Precision.HIGHEST


def _a_body(x_ref, sig_ref, posm_ref, a_ref):
    x = x_ref[...]
    sig_ref[0:3125, :] = jax.nn.sigmoid(x)
    sig_ref[3125:ROWS, :] = jnp.zeros((ROWS - 3125, 128), jnp.float32)
    sb = lax.bitcast_convert_type(sig_ref[...], jnp.int32)

    def cond(c):
        return c[1] - c[0] > 1

    def body(c):
        lo, hi = c
        mid = (lo + hi) // 2
        cnt = jnp.sum(jnp.where(sb > mid, 1, 0))
        return jnp.where(cnt >= N, mid, lo), jnp.where(cnt >= N, hi, mid)

    _, pstar = lax.while_loop(cond, body, (jnp.int32(0), jnp.int32(0x3F800000)))

    gm = jnp.where(sb > pstar, 1.0, 0.0)
    em = jnp.where(sb == pstar, 1.0, 0.0)
    ri = lax.broadcasted_iota(jnp.int32, (BLK, BLK), 0)
    ci = lax.broadcasted_iota(jnp.int32, (BLK, BLK), 1)
    mstrict = jnp.where(ri < ci, 1.0, 0.0)
    gex = lax.dot_general(gm, mstrict, (((1,), (0,)), ((), ())),
                          preferred_element_type=jnp.float32)
    eex = lax.dot_general(em, mstrict, (((1,), (0,)), ((), ())),
                          preferred_element_type=jnp.float32)
    a_ref[:, 0:1] = jnp.sum(gm, axis=1, keepdims=True)
    a_ref[:, 1:2] = jnp.sum(em, axis=1, keepdims=True)

    rb_r = lax.broadcasted_iota(jnp.int32, (RB, RB), 0)
    rb_c = lax.broadcasted_iota(jnp.int32, (RB, RB), 1)
    lstrict = jnp.where(rb_c < rb_r, 1.0, 0.0)
    rowi = lax.broadcasted_iota(jnp.int32, (RB, 1), 0)

    def rp_step(b, c):
        cg, ce = c
        gt = a_ref[pl.ds(b * RB, RB), 0:1]
        et = a_ref[pl.ds(b * RB, RB), 1:2]
        a_ref[pl.ds(b * RB, RB), 2:3] = lax.dot_general(
            lstrict, gt, (((1,), (0,)), ((), ())),
            preferred_element_type=jnp.float32) + cg
        a_ref[pl.ds(b * RB, RB), 3:4] = lax.dot_general(
            lstrict, et, (((1,), (0,)), ((), ())),
            preferred_element_type=jnp.float32) + ce
        return cg + jnp.sum(gt), ce + jnp.sum(et)

    m, _ = lax.fori_loop(0, ROWS // RB, rp_step, (jnp.float32(0.0), jnp.float32(0.0)))
    r = jnp.float32(N) - m

    pos_g = a_ref[:, 2:3] + gex
    pos_e = a_ref[:, 3:4] + eex
    pos = jnp.where(gm > 0.5, pos_g,
                    jnp.where(em > 0.5,
                              jnp.where(pos_e < r, m + pos_e, -1.0),
                              -1.0))
    posm_ref[...] = pos.astype(jnp.int32)


def _run_a(x2d, interpret=False):
    return pl.pallas_call(
        _a_body,
        out_shape=(jax.ShapeDtypeStruct((ROWS, 128), jnp.float32),
                   jax.ShapeDtypeStruct((ROWS, 128), jnp.int32)),
        scratch_shapes=[pltpu.VMEM((ROWS, 8), jnp.float32)],
        interpret=interpret,
    )(x2d)


def _b_body(scores_hbm, posm_hbm, vals_hbm, idx_hbm, sc_v, pos_v, lv, li):
    w = lax.axis_index("s") * 2 + lax.axis_index("c")
    base = w * PW
    pltpu.sync_copy(scores_hbm.at[pl.ds(base, PW)], sc_v)
    pltpu.sync_copy(posm_hbm.at[pl.ds(base, PW)], pos_v)
    lanes = lax.iota(jnp.int32, 16)

    def zstep(i, c):
        lv[pl.ds(i * 16, 16)] = jnp.zeros((16,), jnp.float32)
        li[pl.ds(i * 16, 16)] = jnp.zeros((16,), jnp.int32)
        return c

    lax.fori_loop(0, NPAD // 16, zstep, jnp.int32(0))

    def step(i, c):
        v = sc_v[pl.ds(i * 16, 16)]
        p = pos_v[pl.ds(i * 16, 16)]
        flat = base + i * 16 + lanes
        msk = p >= 0
        pc = jnp.where(msk, p, 0)
        plsc.store_scatter(lv, [pc], v, mask=msk)
        plsc.store_scatter(li, [pc], flat, mask=msk)
        return c

    lax.fori_loop(0, PW // 16, step, jnp.int32(0))
    pltpu.sync_copy(lv, vals_hbm.at[w])
    pltpu.sync_copy(li, idx_hbm.at[w])


def _run_b(scores_flat, posm_flat):
    mesh = plsc.VectorSubcoreMesh(core_axis_name="c", subcore_axis_name="s")
    f = pl.kernel(
        _b_body,
        out_type=(jax.ShapeDtypeStruct((NW, NPAD), jnp.float32),
                  jax.ShapeDtypeStruct((NW, NPAD), jnp.int32)),
        mesh=mesh,
        compiler_params=pltpu.CompilerParams(needs_layout_passes=False),
        scratch_types=[
            pltpu.VMEM((PW,), jnp.float32),
            pltpu.VMEM((PW,), jnp.int32),
            pltpu.VMEM((NPAD,), jnp.float32),
            pltpu.VMEM((NPAD,), jnp.int32),
        ],
    )
    return f(scores_flat, posm_flat)


def _row2col(v, eye):
    return jnp.sum(eye * v, axis=1, keepdims=True)


def _col2row(v, eye):
    return jnp.sum(eye * v, axis=0, keepdims=True)


def _c_body(vals2_ref, idx2_ref, boxpay_ref, dets_ref, lab_ref, t_ref, s_ref):
    col = lax.broadcasted_iota(jnp.int32, (1, NPAD), 1)
    ri = lax.broadcasted_iota(jnp.int32, (CBLK, CBLK), 0)
    ci = lax.broadcasted_iota(jnp.int32, (CBLK, CBLK), 1)
    eye = jnp.where(ri == ci, 1.0, 0.0)
    real = col < N

    vrow = jnp.sum(vals2_ref[...], axis=0, keepdims=True)
    irow = jnp.sum(idx2_ref[...], axis=0, keepdims=True)
    vrow = jnp.where(real, vrow, 0.0)
    irow = jnp.where(real, irow, 0)
    l2 = (irow >> 16).astype(jnp.float32)
    l1 = ((irow >> 8) & 255).astype(jnp.float32)
    l0 = (irow & 255).astype(jnp.float32)
    s_ref[5:6, :] = vrow
    colf = col.astype(jnp.float32)

    def rank_step(i, c):
        base = i * CBLK
        vb = _row2col(s_ref[5:6, pl.ds(base, CBLK)], eye)
        jg = (base + lax.broadcasted_iota(jnp.int32, (CBLK, 1), 0)).astype(jnp.float32)
        gt = jnp.where(vrow > vb, 1.0, 0.0)
        eq = jnp.where(vrow == vb, 1.0, 0.0) * jnp.where(colf < jg, 1.0, 0.0)
        rk = jnp.sum(gt + eq, axis=1, keepdims=True)
        t_ref[6:7, pl.ds(base, CBLK)] = _col2row(rk, eye)
        return c

    lax.fori_loop(0, CNB, rank_step, jnp.int32(0))

    rank_row = t_ref[6:7, :]
    boxpay = boxpay_ref[...]

    def matvec(oh, row):
        return lax.dot_general(oh, row, (((1,), (1,)), ((), ())),
                               precision=HIGH,
                               preferred_element_type=jnp.float32)

    def perm_step(i, c):
        base = i * CBLK
        jout = (base + lax.broadcasted_iota(jnp.int32, (CBLK, 1), 0)).astype(jnp.float32)
        oh = jnp.where(rank_row == jout, 1.0, 0.0)
        sv = matvec(oh, vrow)
        idx = (matvec(oh, l2) * 65536.0 + matvec(oh, l1) * 256.0
               + matvec(oh, l0)).astype(jnp.int32)
        prop = idx // K
        label = idx - prop * K
        oh2 = jnp.where(colf == prop.astype(jnp.float32), 1.0, 0.0)
        bl = lax.dot_general(oh2, boxpay, (((1,), (0,)), ((), ())),
                             precision=HIGH,
                             preferred_element_type=jnp.float32)

        def coord(cix):
            l3i = bl[:, 4 * cix + 0:4 * cix + 1].astype(jnp.int32)
            l2i = bl[:, 4 * cix + 1:4 * cix + 2].astype(jnp.int32)
            l1i = bl[:, 4 * cix + 2:4 * cix + 3].astype(jnp.int32)
            l0i = bl[:, 4 * cix + 3:4 * cix + 4].astype(jnp.int32)
            bits = l3i * 16777216 + l2i * 65536 + l1i * 256 + l0i
            return lax.bitcast_convert_type(bits, jnp.float32)

        cx, cy, bw, bh = coord(0), coord(1), coord(2), coord(3)
        x1 = (cx - 0.5 * bw) * IMG
        y1 = (cy - 0.5 * bh) * IMG
        x2 = (cx + 0.5 * bw) * IMG
        y2 = (cy + 0.5 * bh) * IMG
        t_ref[0:1, pl.ds(base, CBLK)] = _col2row(x1, eye)
        t_ref[1:2, pl.ds(base, CBLK)] = _col2row(y1, eye)
        t_ref[2:3, pl.ds(base, CBLK)] = _col2row(x2, eye)
        t_ref[3:4, pl.ds(base, CBLK)] = _col2row(y2, eye)
        t_ref[4:5, pl.ds(base, CBLK)] = _col2row(label.astype(jnp.float32), eye)
        t_ref[5:6, pl.ds(base, CBLK)] = _col2row(sv, eye)
        return c

    lax.fori_loop(0, CNB, perm_step, jnp.int32(0))

    x1 = t_ref[0:1, :]
    y1 = t_ref[1:2, :]
    x2 = t_ref[2:3, :]
    y2 = t_ref[3:4, :]
    labels_f = t_ref[4:5, :]
    vals_s = t_ref[5:6, :]
    neg = jnp.float32(-1e30)
    mx = jnp.max(jnp.where(real, jnp.maximum(jnp.maximum(x1, y1),
                                             jnp.maximum(x2, y2)), neg))
    maxc = mx + 1.0
    off = labels_f * maxc
    ox1 = x1 + off
    oy1 = y1 + off
    ox2 = x2 + off
    oy2 = y2 + off
    area = (ox2 - ox1) * (oy2 - oy1)
    s_ref[0:1, :] = ox1
    s_ref[1:2, :] = oy1
    s_ref[2:3, :] = ox2
    s_ref[3:4, :] = oy2
    s_ref[4:5, :] = area
    t_ref[7:8, :] = jnp.zeros((1, NPAD), jnp.float32)

    def block_step(i, carry):
        base = i * CBLK
        rx1 = s_ref[0:1, pl.ds(base, CBLK)]
        ry1 = s_ref[1:2, pl.ds(base, CBLK)]
        rx2 = s_ref[2:3, pl.ds(base, CBLK)]
        ry2 = s_ref[3:4, pl.ds(base, CBLK)]
        rar = s_ref[4:5, pl.ds(base, CBLK)]
        bx1 = _row2col(rx1, eye)
        by1 = _row2col(ry1, eye)
        bx2 = _row2col(rx2, eye)
        by2 = _row2col(ry2, eye)
        bar = _row2col(rar, eye)

        def over_f(ax1, ay1, ax2, ay2, aar):
            ix1 = jnp.maximum(bx1, ax1)
            iy1 = jnp.maximum(by1, ay1)
            ix2 = jnp.minimum(bx2, ax2)
            iy2 = jnp.minimum(by2, ay2)
            inter = jnp.maximum(ix2 - ix1, 0.0) * jnp.maximum(iy2 - iy1, 0.0)
            iou = inter / (bar + aar - inter + 1e-9)
            return jnp.where(iou > NMS_THRESH, 1.0, 0.0)

        over_all = over_f(ox1, oy1, ox2, oy2, area)
        keep = t_ref[7:8, :]
        prior = jnp.where(col < base, keep, 0.0)
        supp0 = jnp.sum(over_all * prior, axis=1, keepdims=True)
        e_col = 1.0 - jnp.minimum(supp0, 1.0)
        tri = jnp.where(ri < ci, 1.0, 0.0)
        ob = over_f(rx1, ry1, rx2, ry2, rar) * tri

        def fp_cond(c):
            return c[1] > 0.5

        def fp_body(c):
            kb, _ = c
            sup = jnp.sum(ob * kb, axis=0, keepdims=True)
            sup_col = _row2col(jnp.minimum(sup, 1.0), eye)
            nk = e_col * (1.0 - jnp.minimum(sup_col, 1.0))
            return nk, jnp.sum(jnp.abs(nk - kb))

        kb, _ = lax.while_loop(fp_cond, fp_body, (e_col, jnp.float32(1.0)))
        t_ref[7:8, pl.ds(base, CBLK)] = _col2row(kb, eye)
        return carry

    lax.fori_loop(0, CNB, block_step, jnp.int32(0))
    keep = jnp.where(real, t_ref[7:8, :], 0.0)

    c0 = jnp.zeros((1, NPAD), jnp.float32)
    dets_ref[0:1, :] = jnp.clip(x1, 0.0, IMG) * keep
    dets_ref[1:2, :] = jnp.clip(y1, 0.0, IMG) * keep
    dets_ref[2:3, :] = jnp.clip(x2, 0.0, IMG) * keep
    dets_ref[3:4, :] = jnp.clip(y2, 0.0, IMG) * keep
    dets_ref[4:5, :] = vals_s * keep
    dets_ref[5:6, :] = c0
    dets_ref[6:7, :] = c0
    dets_ref[7:8, :] = c0
    lab_ref[0:1, :] = labels_f


def _run_c(vals2, idx2, boxpay, interpret=False):
    return pl.pallas_call(
        _c_body,
        out_shape=(jax.ShapeDtypeStruct((8, NPAD), jnp.float32),
                   jax.ShapeDtypeStruct((1, NPAD), jnp.float32)),
        scratch_shapes=[pltpu.VMEM((8, NPAD), jnp.float32),
                        pltpu.VMEM((8, NPAD), jnp.float32)],
        interpret=interpret,
    )(vals2, idx2, boxpay)


def _boxpay(box_pred):
    bbits = lax.bitcast_convert_type(box_pred[0], jnp.int32)
    limbs = [((bbits[:, c] >> s) & 255).astype(jnp.float32)
             for c in range(4) for s in (24, 16, 8, 0)]
    return jnp.pad(jnp.stack(limbs, axis=1), ((0, NPAD - N), (0, 0)))


@jax.jit
def _pipeline(box_cls, box_pred):
    x2d = box_cls[0].reshape(3125, 128)
    sig, posm = _run_a(x2d)
    vals2, idx2 = _run_b(sig.reshape(TOT), posm.reshape(TOT))
    dets8, lab = _run_c(vals2, idx2, _boxpay(box_pred))
    dets = dets8[:5, :N].T
    labels = lab[0, :N].astype(jnp.int32)
    return dets, labels


def kernel(box_cls, box_pred):
    return _pipeline(box_cls, box_pred)


# speedup vs baseline: 1.7011x; 1.0181x over previous
import jax
import jax.numpy as jnp
from jax import lax
from jax.experimental import pallas as pl
from jax.experimental.pallas import tpu as pltpu
from jax.experimental.pallas import tpu_sc as plsc

N = 5000
K = 80
IMG = 640.0
NMS_THRESH = 0.5
NPAD = 5120
BLK = 128
CBLK = 256
CNB = NPAD // CBLK
NBLK = NPAD // BLK
NW = 32
PW = 12544
TOT = NW * PW
ROWS = TOT // 128
RB = 392
HIGH = jax.lax.Precision.HIGHEST


def _a_body(x_ref, sig_ref, posm_ref, a_ref):
    x = x_ref[...]
    sig_ref[0:3125, :] = jax.nn.sigmoid(x)
    sig_ref[3125:ROWS, :] = jnp.zeros((ROWS - 3125, 128), jnp.float32)
    sb = lax.bitcast_convert_type(sig_ref[...], jnp.int32)

    def cond(c):
        return c[1] - c[0] > 1

    def body(c):
        lo, hi = c
        d = hi - lo
        m1 = lo + d // 4
        m2 = lo + d // 2
        m3 = lo + d - d // 4
        c1 = jnp.sum(jnp.where(sb > m1, 1, 0))
        c2 = jnp.sum(jnp.where(sb > m2, 1, 0))
        c3 = jnp.sum(jnp.where(sb > m3, 1, 0))
        lo2 = jnp.where(c3 >= N, m3,
                        jnp.where(c2 >= N, m2,
                                  jnp.where(c1 >= N, m1, lo)))
        hi2 = jnp.where(c1 < N, m1,
                        jnp.where(c2 < N, m2,
                                  jnp.where(c3 < N, m3, hi)))
        return lo2, hi2

    _, pstar = lax.while_loop(cond, body, (jnp.int32(0), jnp.int32(0x3F800000)))

    gm = jnp.where(sb > pstar, 1.0, 0.0)
    em = jnp.where(sb == pstar, 1.0, 0.0)
    ri = lax.broadcasted_iota(jnp.int32, (BLK, BLK), 0)
    ci = lax.broadcasted_iota(jnp.int32, (BLK, BLK), 1)
    mstrict = jnp.where(ri < ci, 1.0, 0.0)
    gex = lax.dot_general(gm, mstrict, (((1,), (0,)), ((), ())),
                          preferred_element_type=jnp.float32)
    eex = lax.dot_general(em, mstrict, (((1,), (0,)), ((), ())),
                          preferred_element_type=jnp.float32)
    a_ref[:, 0:1] = jnp.sum(gm, axis=1, keepdims=True)
    a_ref[:, 1:2] = jnp.sum(em, axis=1, keepdims=True)

    rb_r = lax.broadcasted_iota(jnp.int32, (RB, RB), 0)
    rb_c = lax.broadcasted_iota(jnp.int32, (RB, RB), 1)
    lstrict = jnp.where(rb_c < rb_r, 1.0, 0.0)
    rowi = lax.broadcasted_iota(jnp.int32, (RB, 1), 0)

    def rp_step(b, c):
        cg, ce = c
        gt = a_ref[pl.ds(b * RB, RB), 0:1]
        et = a_ref[pl.ds(b * RB, RB), 1:2]
        a_ref[pl.ds(b * RB, RB), 2:3] = lax.dot_general(
            lstrict, gt, (((1,), (0,)), ((), ())),
            preferred_element_type=jnp.float32) + cg
        a_ref[pl.ds(b * RB, RB), 3:4] = lax.dot_general(
            lstrict, et, (((1,), (0,)), ((), ())),
            preferred_element_type=jnp.float32) + ce
        return cg + jnp.sum(gt), ce + jnp.sum(et)

    m, _ = lax.fori_loop(0, ROWS // RB, rp_step, (jnp.float32(0.0), jnp.float32(0.0)))
    r = jnp.float32(N) - m

    pos_g = a_ref[:, 2:3] + gex
    pos_e = a_ref[:, 3:4] + eex
    pos = jnp.where(gm > 0.5, pos_g,
                    jnp.where(em > 0.5,
                              jnp.where(pos_e < r, m + pos_e, -1.0),
                              -1.0))
    posm_ref[...] = pos.astype(jnp.int32)


def _run_a(x2d, interpret=False):
    return pl.pallas_call(
        _a_body,
        out_shape=(jax.ShapeDtypeStruct((ROWS, 128), jnp.float32),
                   jax.ShapeDtypeStruct((ROWS, 128), jnp.int32)),
        scratch_shapes=[pltpu.VMEM((ROWS, 8), jnp.float32)],
        interpret=interpret,
    )(x2d)


def _b_body(scores_hbm, posm_hbm, vals_hbm, idx_hbm, sc_v, pos_v, lv, li):
    w = lax.axis_index("s") * 2 + lax.axis_index("c")
    base = w * PW
    pltpu.sync_copy(scores_hbm.at[pl.ds(base, PW)], sc_v)
    pltpu.sync_copy(posm_hbm.at[pl.ds(base, PW)], pos_v)
    lanes = lax.iota(jnp.int32, 16)

    def zstep(i, c):
        lv[pl.ds(i * 16, 16)] = jnp.zeros((16,), jnp.float32)
        li[pl.ds(i * 16, 16)] = jnp.zeros((16,), jnp.int32)
        return c

    lax.fori_loop(0, NPAD // 16, zstep, jnp.int32(0))

    def step(i, c):
        v = sc_v[pl.ds(i * 16, 16)]
        p = pos_v[pl.ds(i * 16, 16)]
        flat = base + i * 16 + lanes
        msk = p >= 0
        pc = jnp.where(msk, p, 0)
        plsc.store_scatter(lv, [pc], v, mask=msk)
        plsc.store_scatter(li, [pc], flat, mask=msk)
        return c

    lax.fori_loop(0, PW // 16, step, jnp.int32(0))
    pltpu.sync_copy(lv, vals_hbm.at[w])
    pltpu.sync_copy(li, idx_hbm.at[w])


def _run_b(scores_flat, posm_flat):
    mesh = plsc.VectorSubcoreMesh(core_axis_name="c", subcore_axis_name="s")
    f = pl.kernel(
        _b_body,
        out_type=(jax.ShapeDtypeStruct((NW, NPAD), jnp.float32),
                  jax.ShapeDtypeStruct((NW, NPAD), jnp.int32)),
        mesh=mesh,
        compiler_params=pltpu.CompilerParams(needs_layout_passes=False),
        scratch_types=[
            pltpu.VMEM((PW,), jnp.float32),
            pltpu.VMEM((PW,), jnp.int32),
            pltpu.VMEM((NPAD,), jnp.float32),
            pltpu.VMEM((NPAD,), jnp.int32),
        ],
    )
    return f(scores_flat, posm_flat)


def _row2col(v, eye):
    return jnp.sum(eye * v, axis=1, keepdims=True)


def _col2row(v, eye):
    return jnp.sum(eye * v, axis=0, keepdims=True)


def _c_body(vals2_ref, idx2_ref, boxpay_ref, dets_ref, lab_ref, t_ref, s_ref):
    col = lax.broadcasted_iota(jnp.int32, (1, NPAD), 1)
    ri = lax.broadcasted_iota(jnp.int32, (CBLK, CBLK), 0)
    ci = lax.broadcasted_iota(jnp.int32, (CBLK, CBLK), 1)
    eye = jnp.where(ri == ci, 1.0, 0.0)
    real = col < N

    vrow = jnp.sum(vals2_ref[...], axis=0, keepdims=True)
    irow = jnp.sum(idx2_ref[...], axis=0, keepdims=True)
    vrow = jnp.where(real, vrow, 0.0)
    irow = jnp.where(real, irow, 0)
    l2 = (irow >> 16).astype(jnp.float32)
    l1 = ((irow >> 8) & 255).astype(jnp.float32)
    l0 = (irow & 255).astype(jnp.float32)
    s_ref[5:6, :] = vrow
    colf = col.astype(jnp.float32)

    def rank_step(i, c):
        base = i * CBLK
        vb = _row2col(s_ref[5:6, pl.ds(base, CBLK)], eye)
        jg = (base + lax.broadcasted_iota(jnp.int32, (CBLK, 1), 0)).astype(jnp.float32)
        gt = jnp.where(vrow > vb, 1.0, 0.0)
        eq = jnp.where(vrow == vb, 1.0, 0.0) * jnp.where(colf < jg, 1.0, 0.0)
        rk = jnp.sum(gt + eq, axis=1, keepdims=True)
        t_ref[6:7, pl.ds(base, CBLK)] = _col2row(rk, eye)
        return c

    lax.fori_loop(0, CNB, rank_step, jnp.int32(0))

    rank_row = t_ref[6:7, :]
    boxpay = boxpay_ref[...]

    def matvec(oh, row):
        return lax.dot_general(oh, row, (((1,), (1,)), ((), ())),
                               precision=HIGH,
                               preferred_element_type=jnp.float32)

    def perm_step(i, c):
        base = i * CBLK
        jout = (base + lax.broadcasted_iota(jnp.int32, (CBLK, 1), 0)).astype(jnp.float32)
        oh = jnp.where(rank_row == jout, 1.0, 0.0)
        sv = matvec(oh, vrow)
        idx = (matvec(oh, l2) * 65536.0 + matvec(oh, l1) * 256.0
               + matvec(oh, l0)).astype(jnp.int32)
        prop = idx // K
        label = idx - prop * K
        oh2 = jnp.where(colf == prop.astype(jnp.float32), 1.0, 0.0)
        bl = lax.dot_general(oh2, boxpay, (((1,), (0,)), ((), ())),
                             precision=HIGH,
                             preferred_element_type=jnp.float32)

        def coord(cix):
            l3i = bl[:, 4 * cix + 0:4 * cix + 1].astype(jnp.int32)
            l2i = bl[:, 4 * cix + 1:4 * cix + 2].astype(jnp.int32)
            l1i = bl[:, 4 * cix + 2:4 * cix + 3].astype(jnp.int32)
            l0i = bl[:, 4 * cix + 3:4 * cix + 4].astype(jnp.int32)
            bits = l3i * 16777216 + l2i * 65536 + l1i * 256 + l0i
            return lax.bitcast_convert_type(bits, jnp.float32)

        cx, cy, bw, bh = coord(0), coord(1), coord(2), coord(3)
        x1 = (cx - 0.5 * bw) * IMG
        y1 = (cy - 0.5 * bh) * IMG
        x2 = (cx + 0.5 * bw) * IMG
        y2 = (cy + 0.5 * bh) * IMG
        t_ref[0:1, pl.ds(base, CBLK)] = _col2row(x1, eye)
        t_ref[1:2, pl.ds(base, CBLK)] = _col2row(y1, eye)
        t_ref[2:3, pl.ds(base, CBLK)] = _col2row(x2, eye)
        t_ref[3:4, pl.ds(base, CBLK)] = _col2row(y2, eye)
        t_ref[4:5, pl.ds(base, CBLK)] = _col2row(label.astype(jnp.float32), eye)
        t_ref[5:6, pl.ds(base, CBLK)] = _col2row(sv, eye)
        return c

    lax.fori_loop(0, CNB, perm_step, jnp.int32(0))

    x1 = t_ref[0:1, :]
    y1 = t_ref[1:2, :]
    x2 = t_ref[2:3, :]
    y2 = t_ref[3:4, :]
    labels_f = t_ref[4:5, :]
    vals_s = t_ref[5:6, :]
    neg = jnp.float32(-1e30)
    mx = jnp.max(jnp.where(real, jnp.maximum(jnp.maximum(x1, y1),
                                             jnp.maximum(x2, y2)), neg))
    maxc = mx + 1.0
    off = labels_f * maxc
    ox1 = x1 + off
    oy1 = y1 + off
    ox2 = x2 + off
    oy2 = y2 + off
    area = (ox2 - ox1) * (oy2 - oy1)
    s_ref[0:1, :] = ox1
    s_ref[1:2, :] = oy1
    s_ref[2:3, :] = ox2
    s_ref[3:4, :] = oy2
    s_ref[4:5, :] = area
    t_ref[7:8, :] = jnp.zeros((1, NPAD), jnp.float32)

    def block_step(i, carry):
        base = i * CBLK
        rx1 = s_ref[0:1, pl.ds(base, CBLK)]
        ry1 = s_ref[1:2, pl.ds(base, CBLK)]
        rx2 = s_ref[2:3, pl.ds(base, CBLK)]
        ry2 = s_ref[3:4, pl.ds(base, CBLK)]
        rar = s_ref[4:5, pl.ds(base, CBLK)]
        bx1 = _row2col(rx1, eye)
        by1 = _row2col(ry1, eye)
        bx2 = _row2col(rx2, eye)
        by2 = _row2col(ry2, eye)
        bar = _row2col(rar, eye)

        def over_f(ax1, ay1, ax2, ay2, aar):
            ix1 = jnp.maximum(bx1, ax1)
            iy1 = jnp.maximum(by1, ay1)
            ix2 = jnp.minimum(bx2, ax2)
            iy2 = jnp.minimum(by2, ay2)
            inter = jnp.maximum(ix2 - ix1, 0.0) * jnp.maximum(iy2 - iy1, 0.0)
            iou = inter / (bar + aar - inter + 1e-9)
            return jnp.where(iou > NMS_THRESH, 1.0, 0.0)

        over_all = over_f(ox1, oy1, ox2, oy2, area)
        keep = t_ref[7:8, :]
        prior = jnp.where(col < base, keep, 0.0)
        supp0 = jnp.sum(over_all * prior, axis=1, keepdims=True)
        e_col = 1.0 - jnp.minimum(supp0, 1.0)
        tri = jnp.where(ri < ci, 1.0, 0.0)
        ob = over_f(rx1, ry1, rx2, ry2, rar) * tri

        def fp_cond(c):
            return c[1] > 0.5

        def fp_body(c):
            kb, _ = c
            sup = jnp.sum(ob * kb, axis=0, keepdims=True)
            sup_col = _row2col(jnp.minimum(sup, 1.0), eye)
            nk = e_col * (1.0 - jnp.minimum(sup_col, 1.0))
            return nk, jnp.sum(jnp.abs(nk - kb))

        kb, _ = lax.while_loop(fp_cond, fp_body, (e_col, jnp.float32(1.0)))
        t_ref[7:8, pl.ds(base, CBLK)] = _col2row(kb, eye)
        return carry

    lax.fori_loop(0, CNB, block_step, jnp.int32(0))
    keep = jnp.where(real, t_ref[7:8, :], 0.0)

    c0 = jnp.zeros((1, NPAD), jnp.float32)
    dets_ref[0:1, :] = jnp.clip(x1, 0.0, IMG) * keep
    dets_ref[1:2, :] = jnp.clip(y1, 0.0, IMG) * keep
    dets_ref[2:3, :] = jnp.clip(x2, 0.0, IMG) * keep
    dets_ref[3:4, :] = jnp.clip(y2, 0.0, IMG) * keep
    dets_ref[4:5, :] = vals_s * keep
    dets_ref[5:6, :] = c0
    dets_ref[6:7, :] = c0
    dets_ref[7:8, :] = c0
    lab_ref[0:1, :] = labels_f


def _run_c(vals2, idx2, boxpay, interpret=False):
    return pl.pallas_call(
        _c_body,
        out_shape=(jax.ShapeDtypeStruct((8, NPAD), jnp.float32),
                   jax.ShapeDtypeStruct((1, NPAD), jnp.float32)),
        scratch_shapes=[pltpu.VMEM((8, NPAD), jnp.float32),
                        pltpu.VMEM((8, NPAD), jnp.float32)],
        interpret=interpret,
    )(vals2, idx2, boxpay)


def _boxpay(box_pred):
    bbits = lax.bitcast_convert_type(box_pred[0], jnp.int32)
    limbs = [((bbits[:, c] >> s) & 255).astype(jnp.float32)
             for c in range(4) for s in (24, 16, 8, 0)]
    return jnp.pad(jnp.stack(limbs, axis=1), ((0, NPAD - N), (0, 0)))


@jax.jit
def _pipeline(box_cls, box_pred):
    x2d = box_cls[0].reshape(3125, 128)
    sig, posm = _run_a(x2d)
    vals2, idx2 = _run_b(sig.reshape(TOT), posm.reshape(TOT))
    dets8, lab = _run_c(vals2, idx2, _boxpay(box_pred))
    dets = dets8[:5, :N].T
    labels = lab[0, :N].astype(jnp.int32)
    return dets, labels


def kernel(box_cls, box_pred):
    return _pipeline(box_cls, box_pred)
